# baseline (speedup 1.0000x reference)
import functools

import jax
import jax.numpy as jnp
from jax import lax
from jax.experimental import pallas as pl
from jax.experimental.pallas import tpu as pltpu
from jax.experimental.pallas import tpu_sc as plsc

EDIM = 64
LANES = 16
NCORES = 2
NSUBCORES = 16
NW = NCORES * NSUBCORES

SEG_W = 20
CHUNK_SEG = 16
CHUNK_IDX = CHUNK_SEG * SEG_W
IDX_MINOR = 64
IDX_ROWS = CHUNK_IDX // IDX_MINOR
PAIR_EDIM = 2 * EDIM


@functools.lru_cache(maxsize=None)
def _sc_embed_sum(nseg):
    segs_per_w = nseg // NW
    nchunk = segs_per_w // CHUNK_SEG
    idx_per_w = segs_per_w * SEG_W

    mesh = plsc.VectorSubcoreMesh(core_axis_name="c", subcore_axis_name="s")

    @functools.partial(
        pl.kernel,
        mesh=mesh,
        out_type=jax.ShapeDtypeStruct((nseg // 2, PAIR_EDIM), jnp.float32),
        scratch_types=[
            pltpu.VMEM((CHUNK_IDX,), jnp.int32),
            pltpu.VMEM((CHUNK_IDX,), jnp.int32),
            pltpu.VMEM((CHUNK_IDX,), jnp.int32),
            pltpu.VMEM((CHUNK_IDX,), jnp.int32),
            pltpu.VMEM((CHUNK_IDX, PAIR_EDIM), jnp.float32),
            pltpu.VMEM((CHUNK_IDX, PAIR_EDIM), jnp.float32),
            pltpu.VMEM((CHUNK_IDX,), jnp.float32),
            pltpu.VMEM((CHUNK_SEG // 2, PAIR_EDIM), jnp.float32),
            pltpu.SemaphoreType.DMA,
            pltpu.SemaphoreType.DMA,
        ],
    )
    def k(words_hbm, table_hbm, out_hbm,
          idx0, idx1, half0, half1, rows0, rows1, maskb, outb, sem0, sem1):
        wid = lax.axis_index("s") * NCORES + lax.axis_index("c")
        seg_base = wid * segs_per_w
        idx_base = wid * idx_per_w

        idxbufs = (idx0, idx1)
        halfbufs = (half0, half1)
        rowbufs = (rows0, rows1)
        sems = (sem0, sem1)

        def stage_and_fire(c, b):
            pltpu.sync_copy(
                words_hbm.at[pl.ds(
                    pl.multiple_of(idx_base + c * CHUNK_IDX, 64), CHUNK_IDX)],
                idxbufs[b],
            )
            for g in range(CHUNK_IDX // LANES):
                v = idxbufs[b][pl.ds(g * LANES, LANES)]
                halfbufs[b][pl.ds(g * LANES, LANES)] = (
                    lax.shift_right_logical(v, 1))
            for j in range(IDX_ROWS):
                pltpu.async_copy(
                    table_hbm.at[halfbufs[b].at[pl.ds(j * IDX_MINOR,
                                                      IDX_MINOR)]],
                    rowbufs[b].at[pl.ds(j * IDX_MINOR, IDX_MINOR)],
                    sems[b],
                )

        def wait_rows(b):
            for j in range(IDX_ROWS):
                pltpu.make_async_copy(
                    table_hbm.at[halfbufs[b].at[pl.ds(j * IDX_MINOR,
                                                      IDX_MINOR)]],
                    rowbufs[b].at[pl.ds(j * IDX_MINOR, IDX_MINOR)],
                    sems[b],
                ).wait()

        def compute(c, b):
            idxb = idxbufs[b]
            halfb = halfbufs[b]
            rows = rowbufs[b]
            for g in range(CHUNK_IDX // LANES):
                v = idxb[pl.ds(g * LANES, LANES)]
                maskb[pl.ds(g * LANES, LANES)] = (
                    jnp.minimum(v, 1).astype(jnp.float32))

            def pair_body(q, carry):
                for sp in range(2):
                    s = 2 * q + sp
                    rb = s * SEG_W
                    mv0 = maskb[pl.ds(rb, LANES)]
                    mv1 = maskb[pl.ds(rb + SEG_W - LANES, LANES)]
                    iv0 = idxb[pl.ds(rb, LANES)]
                    iv1 = idxb[pl.ds(rb + SEG_W - LANES, LANES)]
                    hv0 = halfb[pl.ds(rb, LANES)]
                    hv1 = halfb[pl.ds(rb + SEG_W - LANES, LANES)]
                    accs = [jnp.zeros((LANES,), jnp.float32)
                            for _ in range(EDIM // LANES)]
                    for w in range(SEG_W):
                        if w < LANES:
                            m = mv0[w]
                            par = iv0[w] - 2 * hv0[w]
                        else:
                            m = mv1[w - (SEG_W - LANES)]
                            par = iv1[w - (SEG_W - LANES)] - 2 * hv1[
                                w - (SEG_W - LANES)]
                        base = par * EDIM
                        for d in range(EDIM // LANES):
                            accs[d] = accs[d] + rows[
                                rb + w, pl.ds(base + d * LANES, LANES)] * m
                    for d in range(EDIM // LANES):
                        outb[q, pl.ds(sp * EDIM + d * LANES, LANES)] = accs[d]
                return carry

            lax.fori_loop(0, CHUNK_SEG // 2, pair_body, 0)
            pltpu.sync_copy(
                outb,
                out_hbm.at[pl.ds(
                    pl.multiple_of((seg_base + c * CHUNK_SEG) // 2, 8),
                    CHUNK_SEG // 2)])

        stage_and_fire(0, 0)

        def outer(t, carry):
            for b in range(2):
                c = 2 * t + b

                @pl.when(c + 1 < nchunk)
                def _():
                    stage_and_fire(c + 1, 1 - b)

                wait_rows(b)
                compute(c, b)
            return carry

        lax.fori_loop(0, nchunk // 2, outer, 0)

    return k


def kernel(words, table):
    b, s, w = words.shape
    nrow, edim = table.shape
    assert w == SEG_W and edim == EDIM and nrow % 2 == 0
    nseg = b * s
    assert nseg % (NW * CHUNK_SEG) == 0
    assert CHUNK_IDX % IDX_MINOR == 0
    flat = words.astype(jnp.int32).reshape(nseg * w)
    pairs = table.reshape(nrow // 2, PAIR_EDIM)
    out = _sc_embed_sum(nseg)(flat, pairs)
    return out.reshape(b, s, edim)
